# baseline (speedup 1.0000x reference)
import functools

import jax
import jax.numpy as jnp
from jax import lax
from jax.experimental import pallas as pl
from jax.experimental.pallas import tpu as pltpu
from jax.experimental.pallas import tpu_sc as plsc

_S_SC = 512
_BS = 512
_CH = 4


def _tc_body(pe_ref, x_ref, o_ref):
    o_ref[...] = x_ref[...] + pe_ref[...]


def _make_sc(B, S, D, S_tc):
    NC, NS = 2, 16
    NW = NC * NS
    rows_w = _S_SC // NW
    n_ch = rows_w // _CH
    mesh = plsc.VectorSubcoreMesh(
        core_axis_name="c", subcore_axis_name="s", num_cores=NC, num_subcores=NS
    )

    @functools.partial(
        pl.kernel,
        mesh=mesh,
        out_type=jax.ShapeDtypeStruct((B * _S_SC, D), jnp.float32),
        scratch_types=[
            pltpu.VMEM((B, 2, _CH, D), jnp.float32),
            pltpu.VMEM((2, _CH, D), jnp.float32),
            pltpu.SemaphoreType.DMA((B, 2)),
            pltpu.SemaphoreType.DMA((B, 2)),
            pltpu.SemaphoreType.DMA((2,)),
        ],
    )
    def k(x_hbm, pe_hbm, out_hbm, x_buf, pe_buf, in_sem, out_sem, pe_sem):
        wid = lax.axis_index("s") * NC + lax.axis_index("c")
        base0 = wid * rows_w

        def x_rows(c, b):
            return x_hbm.at[pl.ds(b * S + S_tc + base0 + c * _CH, _CH)]

        def out_rows(c, b):
            return out_hbm.at[pl.ds(b * _S_SC + base0 + c * _CH, _CH)]

        def pe_rows(c):
            return pe_hbm.at[pl.ds(S_tc + base0 + c * _CH, _CH)]

        pltpu.async_copy(pe_rows(0), pe_buf.at[0], pe_sem.at[0])
        for b in range(B):
            pltpu.async_copy(x_rows(0, b), x_buf.at[b, 0], in_sem.at[b, 0])

        def chunk_body(c, _):
            p = lax.rem(c, 2)
            pn = lax.rem(c + 1, 2)
            pltpu.make_async_copy(pe_rows(c), pe_buf.at[p], pe_sem.at[p]).wait()

            @pl.when(c + 1 < n_ch)
            def _():
                pltpu.async_copy(pe_rows(c + 1), pe_buf.at[pn], pe_sem.at[pn])

            for b in range(B):
                pltpu.make_async_copy(
                    x_rows(c, b), x_buf.at[b, p], in_sem.at[b, p]
                ).wait()

                @pl.when(c + 1 < n_ch)
                def _():
                    @pl.when(c > 0)
                    def _():
                        pltpu.make_async_copy(
                            x_buf.at[b, pn], out_rows(c - 1, b), out_sem.at[b, pn]
                        ).wait()

                    pltpu.async_copy(
                        x_rows(c + 1, b), x_buf.at[b, pn], in_sem.at[b, pn]
                    )

                for i in range(_CH):

                    @plsc.parallel_loop(0, D // 16, unroll=8)
                    def _add(j):
                        sl = pl.ds(j * 16, 16)
                        plsc.addupdate(x_buf.at[b, p, i, sl], pe_buf[p, i, sl])

                pltpu.async_copy(x_buf.at[b, p], out_rows(c, b), out_sem.at[b, p])
            return 0

        lax.fori_loop(0, n_ch, chunk_body, 0)

        pl_last = (n_ch - 1) % 2
        for b in range(B):
            pltpu.make_async_copy(
                x_buf.at[b, pl_last],
                out_rows(n_ch - 1, b),
                out_sem.at[b, pl_last],
            ).wait()

    return k


def kernel(x, abs_pe):
    B, S, D = x.shape
    S_tc = S - _S_SC
    nsb = S_tc // _BS
    x2 = x.reshape(B * S, D)
    pe2 = abs_pe.reshape(abs_pe.shape[1], D)

    tc_out = pl.pallas_call(
        _tc_body,
        grid=(nsb, B),
        in_specs=[
            pl.BlockSpec((_BS, D), lambda s, b: (s, 0)),
            pl.BlockSpec((_BS, D), lambda s, b: (b * (S // _BS) + s, 0)),
        ],
        out_specs=pl.BlockSpec((_BS, D), lambda s, b: (b * (S // _BS) + s, 0)),
        out_shape=jax.ShapeDtypeStruct((B * S, D), x.dtype),
        compiler_params=pltpu.CompilerParams(
            dimension_semantics=("arbitrary", "arbitrary"),
        ),
    )(pe2, x2)

    sc_out = _make_sc(B, S, D, S_tc)(x2, pe2)

    out = lax.dynamic_update_slice(
        tc_out.reshape(B, S, D), sc_out.reshape(B, _S_SC, D), (0, S_tc, 0)
    )
    return out


# speedup vs baseline: 1.3436x; 1.3436x over previous
import jax
import jax.numpy as jnp
from jax.experimental import pallas as pl
from jax.experimental.pallas import tpu as pltpu

_BS = 1024


def _body(pe_ref, x_ref, o_ref):
    o_ref[...] = x_ref[...] + pe_ref[...]


def kernel(x, abs_pe):
    B, S, D = x.shape
    nsb = S // _BS
    x2 = x.reshape(B * S, D)
    pe2 = abs_pe.reshape(abs_pe.shape[1], D)
    grid = (nsb, B)
    out = pl.pallas_call(
        _body,
        grid=grid,
        in_specs=[
            pl.BlockSpec((_BS, D), lambda s, b: (s, 0)),
            pl.BlockSpec((_BS, D), lambda s, b: (b * nsb + s, 0)),
        ],
        out_specs=pl.BlockSpec((_BS, D), lambda s, b: (b * nsb + s, 0)),
        out_shape=jax.ShapeDtypeStruct((B * S, D), x.dtype),
        compiler_params=pltpu.CompilerParams(
            dimension_semantics=("arbitrary", "arbitrary"),
        ),
    )(pe2, x2)
    return out.reshape(B, S, D)
